# baseline (speedup 1.0000x reference)
import functools

import jax
import jax.numpy as jnp
from jax import lax
from jax.experimental import pallas as pl
from jax.experimental.pallas import tpu as pltpu
from jax.experimental.pallas import tpu_sc as plsc

B = 1024
L = 200
DIM = 64

NC = 2
NS = 16
NW = NC * NS

CHUNK = 128
TOTAL = B * L
NCHUNKS = TOTAL // CHUNK
CPW = NCHUNKS // NW


def _emb_body(x_hbm, table_hbm, out_hbm, idx_v, rows_v, sem):
    wid = lax.axis_index("s") * NC + lax.axis_index("c")
    base_chunk = wid * CPW

    pltpu.sync_copy(x_hbm.at[wid], idx_v)

    def step(g, carry):
        pltpu.async_copy(table_hbm.at[idx_v.at[g]], rows_v, sem).wait()
        pltpu.sync_copy(rows_v, out_hbm.at[pl.ds((base_chunk + g) * CHUNK, CHUNK)])
        return carry

    lax.fori_loop(0, CPW, step, 0)


@functools.partial(jax.jit, static_argnames=())
def _emb_call(x2d, table):
    mesh = plsc.VectorSubcoreMesh(core_axis_name="c", subcore_axis_name="s")
    fn = pl.kernel(
        _emb_body,
        out_type=jax.ShapeDtypeStruct((TOTAL, DIM), jnp.float32),
        mesh=mesh,
        scratch_types=[
            pltpu.VMEM((CPW, CHUNK), jnp.int32),
            pltpu.VMEM((CHUNK, DIM), jnp.float32),
            pltpu.SemaphoreType.DMA,
        ],
        compiler_params=pltpu.CompilerParams(use_tc_tiling_on_sc=False),
    )
    return fn(x2d, table)


def kernel(x, table):
    x2d = x.reshape(NW, CPW, CHUNK).astype(jnp.int32)
    out = _emb_call(x2d, table)
    return out.reshape(B, L, DIM)


# speedup vs baseline: 1.0449x; 1.0449x over previous
import functools

import jax
import jax.numpy as jnp
from jax import lax
from jax.experimental import pallas as pl
from jax.experimental.pallas import tpu as pltpu
from jax.experimental.pallas import tpu_sc as plsc

B = 1024
L = 200
DIM = 64

NC = 2
NS = 16
NW = NC * NS

CHUNK = 128
TOTAL = B * L
NCHUNKS = TOTAL // CHUNK
CPW = NCHUNKS // NW
GPC = 5
NG = CPW // GPC


def _fire_gathers(table_hbm, idx_v, buf, sem, g):
    for j in range(GPC):
        pltpu.async_copy(table_hbm.at[idx_v.at[g * GPC + j]], buf.at[j], sem)


def _drain_gathers(table_hbm, idx_v, buf, sem):
    for j in range(GPC):
        pltpu.make_async_copy(table_hbm.at[idx_v.at[0]], buf.at[j], sem).wait()


def _fire_writes(out_hbm, buf, sem, base_chunk, g):
    for j in range(GPC):
        dst = out_hbm.at[pl.ds((base_chunk + g * GPC + j) * CHUNK, CHUNK)]
        pltpu.async_copy(buf.at[j], dst, sem)


def _drain_writes(out_hbm, buf, sem):
    for j in range(GPC):
        pltpu.make_async_copy(buf.at[j], out_hbm.at[pl.ds(0, CHUNK)], sem).wait()


def _emb_body(x_hbm, table_hbm, out_hbm, idx_v, buf_a, buf_b, gsem_a, gsem_b,
              wsem_a, wsem_b):
    wid = lax.axis_index("s") * NC + lax.axis_index("c")
    base_chunk = wid * CPW

    pltpu.sync_copy(x_hbm.at[wid], idx_v)

    _fire_gathers(table_hbm, idx_v, buf_a, gsem_a, 0)
    _fire_gathers(table_hbm, idx_v, buf_b, gsem_b, 1)

    def step(tt, carry):
        g_a = 2 * tt
        g_b = g_a + 1
        _drain_gathers(table_hbm, idx_v, buf_a, gsem_a)
        _fire_writes(out_hbm, buf_a, wsem_a, base_chunk, g_a)
        _drain_writes(out_hbm, buf_a, wsem_a)
        _fire_gathers(table_hbm, idx_v, buf_a, gsem_a, g_a + 2)
        _drain_gathers(table_hbm, idx_v, buf_b, gsem_b)
        _fire_writes(out_hbm, buf_b, wsem_b, base_chunk, g_b)
        _drain_writes(out_hbm, buf_b, wsem_b)
        _fire_gathers(table_hbm, idx_v, buf_b, gsem_b, g_b + 2)
        return carry

    lax.fori_loop(0, NG // 2 - 1, step, 0)

    _drain_gathers(table_hbm, idx_v, buf_a, gsem_a)
    _fire_writes(out_hbm, buf_a, wsem_a, base_chunk, NG - 2)
    _drain_gathers(table_hbm, idx_v, buf_b, gsem_b)
    _fire_writes(out_hbm, buf_b, wsem_b, base_chunk, NG - 1)
    _drain_writes(out_hbm, buf_a, wsem_a)
    _drain_writes(out_hbm, buf_b, wsem_b)


@functools.partial(jax.jit, static_argnames=())
def _emb_call(x3d, table):
    mesh = plsc.VectorSubcoreMesh(core_axis_name="c", subcore_axis_name="s")
    fn = pl.kernel(
        _emb_body,
        out_type=jax.ShapeDtypeStruct((TOTAL, DIM), jnp.float32),
        mesh=mesh,
        scratch_types=[
            pltpu.VMEM((CPW, CHUNK), jnp.int32),
            pltpu.VMEM((GPC, CHUNK, DIM), jnp.float32),
            pltpu.VMEM((GPC, CHUNK, DIM), jnp.float32),
            pltpu.SemaphoreType.DMA,
            pltpu.SemaphoreType.DMA,
            pltpu.SemaphoreType.DMA,
            pltpu.SemaphoreType.DMA,
        ],
        compiler_params=pltpu.CompilerParams(use_tc_tiling_on_sc=False),
    )
    return fn(x3d, table)


def kernel(x, table):
    x3d = x.reshape(NW, CPW, CHUNK).astype(jnp.int32)
    out = _emb_call(x3d, table)
    return out.reshape(B, L, DIM)


# speedup vs baseline: 1.0473x; 1.0023x over previous
import functools

import jax
import jax.numpy as jnp
from jax import lax
from jax.experimental import pallas as pl
from jax.experimental.pallas import tpu as pltpu
from jax.experimental.pallas import tpu_sc as plsc

B = 1024
L = 200
DIM = 64

NC = 2
NS = 16
NW = NC * NS

BPW = B // NW
C0 = 128
C1 = L - C0
GPB = 2
NG = BPW // GPB


def _fire_gathers(table_hbm, idx_v, b128, b72, sem, g):
    for j in range(GPB):
        lb = g * GPB + j
        pltpu.async_copy(table_hbm.at[idx_v.at[lb, pl.ds(0, C0)]], b128.at[j], sem)
        pltpu.async_copy(table_hbm.at[idx_v.at[lb, pl.ds(C0, C1)]], b72.at[j], sem)


def _drain_gathers(table_hbm, idx_v, b128, b72, sem):
    for j in range(GPB):
        pltpu.make_async_copy(table_hbm.at[idx_v.at[0, pl.ds(0, C0)]], b128.at[j], sem).wait()
        pltpu.make_async_copy(table_hbm.at[idx_v.at[0, pl.ds(C0, C1)]], b72.at[j], sem).wait()


def _fire_writes(out_hbm, b128, b72, sem, base_b, g):
    for j in range(GPB):
        b = base_b + g * GPB + j
        pltpu.async_copy(b128.at[j], out_hbm.at[b, pl.ds(0, C0)], sem)
        pltpu.async_copy(b72.at[j], out_hbm.at[b, pl.ds(C0, C1)], sem)


def _drain_writes(out_hbm, b128, b72, sem):
    for j in range(GPB):
        pltpu.make_async_copy(b128.at[j], out_hbm.at[0, pl.ds(0, C0)], sem).wait()
        pltpu.make_async_copy(b72.at[j], out_hbm.at[0, pl.ds(C0, C1)], sem).wait()


def _emb_body(x_hbm, table_hbm, out_hbm, idx_v, a128, a72, b128, b72,
              gsem_a, gsem_b, wsem_a, wsem_b):
    wid = lax.axis_index("s") * NC + lax.axis_index("c")
    base_b = wid * BPW

    pltpu.sync_copy(x_hbm.at[pl.ds(base_b, BPW)], idx_v)

    _fire_gathers(table_hbm, idx_v, a128, a72, gsem_a, 0)
    _fire_gathers(table_hbm, idx_v, b128, b72, gsem_b, 1)

    def step(tt, carry):
        g_a = 2 * tt
        g_b = g_a + 1
        _drain_gathers(table_hbm, idx_v, a128, a72, gsem_a)
        _fire_writes(out_hbm, a128, a72, wsem_a, base_b, g_a)
        _drain_writes(out_hbm, a128, a72, wsem_a)
        _fire_gathers(table_hbm, idx_v, a128, a72, gsem_a, g_a + 2)
        _drain_gathers(table_hbm, idx_v, b128, b72, gsem_b)
        _fire_writes(out_hbm, b128, b72, wsem_b, base_b, g_b)
        _drain_writes(out_hbm, b128, b72, wsem_b)
        _fire_gathers(table_hbm, idx_v, b128, b72, gsem_b, g_b + 2)
        return carry

    lax.fori_loop(0, NG // 2 - 1, step, 0)

    _drain_gathers(table_hbm, idx_v, a128, a72, gsem_a)
    _fire_writes(out_hbm, a128, a72, wsem_a, base_b, NG - 2)
    _drain_gathers(table_hbm, idx_v, b128, b72, gsem_b)
    _fire_writes(out_hbm, b128, b72, wsem_b, base_b, NG - 1)
    _drain_writes(out_hbm, a128, a72, wsem_a)
    _drain_writes(out_hbm, b128, b72, wsem_b)


@functools.partial(jax.jit, static_argnames=())
def _emb_call(x, table):
    mesh = plsc.VectorSubcoreMesh(core_axis_name="c", subcore_axis_name="s")
    fn = pl.kernel(
        _emb_body,
        out_type=jax.ShapeDtypeStruct((B, L, DIM), jnp.float32),
        mesh=mesh,
        scratch_types=[
            pltpu.VMEM((BPW, L), jnp.int32),
            pltpu.VMEM((GPB, C0, DIM), jnp.float32),
            pltpu.VMEM((GPB, C1, DIM), jnp.float32),
            pltpu.VMEM((GPB, C0, DIM), jnp.float32),
            pltpu.VMEM((GPB, C1, DIM), jnp.float32),
            pltpu.SemaphoreType.DMA,
            pltpu.SemaphoreType.DMA,
            pltpu.SemaphoreType.DMA,
            pltpu.SemaphoreType.DMA,
        ],
        compiler_params=pltpu.CompilerParams(use_tc_tiling_on_sc=False),
    )
    return fn(x, table)


def kernel(x, table):
    return _emb_call(x.astype(jnp.int32), table)
